# baseline (speedup 1.0000x reference)
import jax
import jax.numpy as jnp
from jax import lax
from jax.experimental import pallas as pl
from jax.experimental.pallas import tpu as pltpu, tpu_sc as plsc

NC = 2
NS = 16
NW = NC * NS
D = 128
CHUNK = 128
NBUF = 4


def _gather_kernel(idx_hbm, table_hbm, out_hbm, idx_v, bufs, gsem, wsem):
    wid = lax.axis_index("s") * NC + lax.axis_index("c")
    j_steps = idx_v.shape[0]
    base = wid * (j_steps * CHUNK)

    pltpu.sync_copy(idx_hbm.at[wid], idx_v)

    gathers = [None] * j_steps
    writes = [None] * j_steps

    def start_gather(j):
        return pltpu.async_copy(
            table_hbm.at[idx_v.at[j]], bufs.at[j % NBUF], gsem)

    for j in range(min(NBUF - 1, j_steps)):
        gathers[j] = start_gather(j)

    for j in range(j_steps):
        nj = j + NBUF - 1
        if nj < j_steps:
            if j >= 1:
                writes[j - 1].wait()
            gathers[nj] = start_gather(nj)
        gathers[j].wait()
        writes[j] = pltpu.async_copy(
            bufs.at[j % NBUF], out_hbm.at[pl.ds(base + j * CHUNK, CHUNK)],
            wsem)

    for j in range(max(0, j_steps - NBUF), j_steps):
        writes[j].wait()


@jax.jit
def _embedding_lookup(idx3, table):
    n_rows = idx3.shape[0] * idx3.shape[1] * idx3.shape[2]
    j_steps = idx3.shape[1]
    mesh = plsc.VectorSubcoreMesh(core_axis_name="c", subcore_axis_name="s")
    return pl.kernel(
        _gather_kernel,
        out_type=jax.ShapeDtypeStruct((n_rows, D), jnp.float32),
        mesh=mesh,
        scratch_types=[
            pltpu.VMEM((j_steps, CHUNK), jnp.int32),
            pltpu.VMEM((NBUF, CHUNK, D), jnp.float32),
            pltpu.SemaphoreType.DMA,
            pltpu.SemaphoreType.DMA,
        ],
    )(idx3, table)


def kernel(batch_original_ids, embedding_weight):
    b, t = batch_original_ids.shape
    idx3 = batch_original_ids.reshape(NW, (b * t) // (NW * CHUNK), CHUNK)
    out = _embedding_lookup(idx3, embedding_weight)
    return out.reshape(b, t, embedding_weight.shape[1])


# speedup vs baseline: 1.5562x; 1.5562x over previous
import jax
import jax.numpy as jnp
from jax import lax
from jax.experimental import pallas as pl
from jax.experimental.pallas import tpu as pltpu, tpu_sc as plsc

NC = 2
NS = 16
NW = NC * NS
D = 128
T = 20
BCHUNK = 4
NBUF = 4


def _gather_kernel(idx_hbm, table_hbm, out_hbm, idx_v, bufs, gsem, wsem):
    wid = lax.axis_index("s") * NC + lax.axis_index("c")
    rows_per_w = idx_v.shape[0] // T
    j_steps = rows_per_w // BCHUNK
    base = wid * rows_per_w

    pltpu.sync_copy(idx_hbm.at[wid], idx_v)

    gathers = [None] * j_steps
    writes = [None] * j_steps

    def start_gather(j):
        return pltpu.async_copy(
            table_hbm.at[idx_v.at[pl.ds(j * BCHUNK * T, BCHUNK * T)]],
            bufs.at[j % NBUF], gsem)

    for j in range(min(NBUF - 1, j_steps)):
        gathers[j] = start_gather(j)

    for j in range(j_steps):
        nj = j + NBUF - 1
        if nj < j_steps:
            if j >= 1:
                writes[j - 1].wait()
            gathers[nj] = start_gather(nj)
        gathers[j].wait()
        writes[j] = pltpu.async_copy(
            bufs.at[j % NBUF].reshape(BCHUNK, T, D),
            out_hbm.at[pl.ds(base + j * BCHUNK, BCHUNK)], wsem)

    for j in range(max(0, j_steps - NBUF), j_steps):
        writes[j].wait()


@jax.jit
def _embedding_lookup(idx2, table):
    b_rows = (idx2.shape[0] * idx2.shape[1]) // T
    mesh = plsc.VectorSubcoreMesh(core_axis_name="c", subcore_axis_name="s")
    return pl.kernel(
        _gather_kernel,
        out_type=jax.ShapeDtypeStruct((b_rows, T, D), jnp.float32),
        mesh=mesh,
        scratch_types=[
            pltpu.VMEM((idx2.shape[1],), jnp.int32),
            pltpu.VMEM((NBUF, BCHUNK * T, D), jnp.float32),
            pltpu.SemaphoreType.DMA,
            pltpu.SemaphoreType.DMA,
        ],
        compiler_params=pltpu.CompilerParams(use_tc_tiling_on_sc=True),
    )(idx2, table)


def kernel(batch_original_ids, embedding_weight):
    b, t = batch_original_ids.shape
    idx2 = batch_original_ids.reshape(NW, (b * t) // NW)
    return _embedding_lookup(idx2, embedding_weight)


# speedup vs baseline: 1.5624x; 1.0040x over previous
import jax
import jax.numpy as jnp
from jax import lax
from jax.experimental import pallas as pl
from jax.experimental.pallas import tpu as pltpu, tpu_sc as plsc

NC = 2
NS = 16
NW = NC * NS
D = 128
T = 20
BCHUNK = 4
NBUF = 4


def _gather_kernel(idx_hbm, table_hbm, out_hbm, idx_v, bufs, gsem, wsem):
    wid = lax.axis_index("s") * NC + lax.axis_index("c")
    rows_per_w = idx_v.shape[0] // T
    j_steps = rows_per_w // BCHUNK
    base = wid * rows_per_w

    pltpu.sync_copy(idx_hbm.at[wid], idx_v)

    gathers = [None] * j_steps
    writes = [None] * j_steps

    def start_gather(j):
        return pltpu.async_copy(
            table_hbm.at[idx_v.at[pl.ds(j * BCHUNK * T, BCHUNK * T)]],
            bufs.at[j % NBUF], gsem)

    for j in range(min(NBUF - 1, j_steps)):
        gathers[j] = start_gather(j)

    for j in range(j_steps):
        nj = j + NBUF - 1
        if nj < j_steps:
            if j >= 1:
                writes[j - 1].wait()
            gathers[nj] = start_gather(nj)
        gathers[j].wait()
        writes[j] = pltpu.async_copy(
            bufs.at[j % NBUF].reshape(BCHUNK, T, D),
            out_hbm.at[pl.ds(base + j * BCHUNK, BCHUNK)], wsem)

    for j in range(max(0, j_steps - NBUF), j_steps):
        writes[j].wait()


@jax.jit
def _embedding_lookup(idx2, table):
    b_rows = (idx2.shape[0] * idx2.shape[1]) // T
    mesh = plsc.VectorSubcoreMesh(core_axis_name="c", subcore_axis_name="s")
    return pl.kernel(
        _gather_kernel,
        out_type=jax.ShapeDtypeStruct((b_rows, T, D), jnp.float32),
        mesh=mesh,
        scratch_types=[
            pltpu.VMEM((idx2.shape[1],), jnp.int32),
            pltpu.VMEM((NBUF, BCHUNK * T, D), jnp.float32),
            pltpu.SemaphoreType.DMA,
            pltpu.SemaphoreType.DMA,
        ],
        compiler_params=pltpu.CompilerParams(
            use_tc_tiling_on_sc=True, needs_layout_passes=True),
    )(idx2, table)


def kernel(batch_original_ids, embedding_weight):
    b, t = batch_original_ids.shape
    idx2 = batch_original_ids.reshape(NW, (b * t) // NW)
    return _embedding_lookup(idx2, embedding_weight)


# speedup vs baseline: 2.5900x; 1.6577x over previous
import jax
import jax.numpy as jnp
from jax import lax
from jax.experimental import pallas as pl
from jax.experimental.pallas import tpu as pltpu, tpu_sc as plsc

NC = 2
NS = 16
NW = NC * NS
D = 128
T = 20
BBLK = 128
NBUF = 4


def _gather_kernel(idx_hbm, table_hbm, out_hbm, idx_v, bufs, gsem, wsem):
    wid = lax.axis_index("s") * NC + lax.axis_index("c")
    base = wid * BBLK

    pltpu.sync_copy(idx_hbm.at[wid], idx_v)

    gathers = [None] * T
    writes = [None] * T

    def start_gather(t):
        return pltpu.async_copy(
            table_hbm.at[idx_v.at[t]], bufs.at[t % NBUF], gsem)

    for t in range(min(NBUF - 1, T)):
        gathers[t] = start_gather(t)

    for t in range(T):
        nt = t + NBUF - 1
        if nt < T:
            if t >= 1:
                writes[t - 1].wait()
            gathers[nt] = start_gather(nt)
        gathers[t].wait()
        writes[t] = pltpu.async_copy(
            bufs.at[t % NBUF], out_hbm.at[t, pl.ds(base, BBLK)], wsem)

    for t in range(max(0, T - NBUF), T):
        writes[t].wait()


@jax.jit
def _embedding_lookup(idx3, table):
    b_rows = idx3.shape[0] * idx3.shape[2]
    mesh = plsc.VectorSubcoreMesh(core_axis_name="c", subcore_axis_name="s")
    out = pl.kernel(
        _gather_kernel,
        out_type=jax.ShapeDtypeStruct((T, b_rows, D), jnp.float32),
        mesh=mesh,
        scratch_types=[
            pltpu.VMEM((T, BBLK), jnp.int32),
            pltpu.VMEM((NBUF, BBLK, D), jnp.float32),
            pltpu.SemaphoreType.DMA,
            pltpu.SemaphoreType.DMA,
        ],
        compiler_params=pltpu.CompilerParams(
            use_tc_tiling_on_sc=True, needs_layout_passes=True),
    )(idx3, table)
    return jnp.transpose(out, (1, 0, 2))


def kernel(batch_original_ids, embedding_weight):
    b, t = batch_original_ids.shape
    idx3 = batch_original_ids.T.reshape(t, NW, b // NW).transpose(1, 0, 2)
    return _embedding_lookup(idx3, embedding_weight)


# speedup vs baseline: 2.5957x; 1.0022x over previous
import jax
import jax.numpy as jnp
from jax import lax
from jax.experimental import pallas as pl
from jax.experimental.pallas import tpu as pltpu, tpu_sc as plsc

NC = 2
NS = 16
NW = NC * NS
D = 128
T = 20
BBLK = 128
NBUF = 6


def _gather_kernel(idx_hbm, table_hbm, out_hbm, idx_v, bufs, gsem, wsem):
    wid = lax.axis_index("s") * NC + lax.axis_index("c")
    base = wid * BBLK

    pltpu.sync_copy(idx_hbm.at[wid], idx_v)

    gathers = [None] * T
    writes = [None] * T

    def start_gather(t):
        return pltpu.async_copy(
            table_hbm.at[idx_v.at[t]], bufs.at[t % NBUF], gsem)

    for t in range(min(NBUF - 1, T)):
        gathers[t] = start_gather(t)

    for t in range(T):
        nt = t + NBUF - 1
        if nt < T:
            if t >= 1:
                writes[t - 1].wait()
            gathers[nt] = start_gather(nt)
        gathers[t].wait()
        writes[t] = pltpu.async_copy(
            bufs.at[t % NBUF], out_hbm.at[t, pl.ds(base, BBLK)], wsem)

    for t in range(max(0, T - NBUF), T):
        writes[t].wait()


@jax.jit
def _embedding_lookup(idx3, table):
    b_rows = idx3.shape[0] * idx3.shape[2]
    mesh = plsc.VectorSubcoreMesh(core_axis_name="c", subcore_axis_name="s")
    out = pl.kernel(
        _gather_kernel,
        out_type=jax.ShapeDtypeStruct((T, b_rows, D), jnp.float32),
        mesh=mesh,
        scratch_types=[
            pltpu.VMEM((T, BBLK), jnp.int32),
            pltpu.VMEM((NBUF, BBLK, D), jnp.float32),
            pltpu.SemaphoreType.DMA,
            pltpu.SemaphoreType.DMA,
        ],
        compiler_params=pltpu.CompilerParams(
            use_tc_tiling_on_sc=True, needs_layout_passes=True),
    )(idx3, table)
    return jnp.transpose(out, (1, 0, 2))


def kernel(batch_original_ids, embedding_weight):
    b, t = batch_original_ids.shape
    idx3 = batch_original_ids.T.reshape(t, NW, b // NW).transpose(1, 0, 2)
    return _embedding_lookup(idx3, embedding_weight)


# speedup vs baseline: 2.6421x; 1.0179x over previous
import jax
import jax.numpy as jnp
from jax import lax
from jax.experimental import pallas as pl
from jax.experimental.pallas import tpu as pltpu, tpu_sc as plsc

NC = 2
NS = 16
NW = NC * NS
D = 128
T = 20
BBLK = 128
NBUF = 6


def _gather_kernel(idx_hbm, table_hbm, out_hbm, idx_v, bufs, gsem, wsem):
    wid = lax.axis_index("s") * NC + lax.axis_index("c")
    base = wid * BBLK

    pltpu.sync_copy(idx_hbm.at[wid], idx_v)

    def wait_gather():
        pltpu.make_async_copy(
            table_hbm.at[pl.ds(0, BBLK)], bufs.at[0], gsem).wait()

    def wait_write():
        pltpu.make_async_copy(
            bufs.at[0], out_hbm.at[0, pl.ds(base, BBLK)], wsem).wait()

    for t in range(NBUF - 1):
        pltpu.async_copy(table_hbm.at[idx_v.at[t]], bufs.at[t], gsem)

    def body(t, _):
        nt = t + NBUF - 1

        @pl.when(jnp.logical_and(nt < T, t >= 1))
        def _():
            wait_write()

        @pl.when(nt < T)
        def _():
            pltpu.async_copy(
                table_hbm.at[idx_v.at[nt]], bufs.at[lax.rem(nt, NBUF)], gsem)

        wait_gather()
        pltpu.async_copy(
            bufs.at[lax.rem(t, NBUF)], out_hbm.at[t, pl.ds(base, BBLK)], wsem)
        return ()

    lax.fori_loop(0, T, body, (), unroll=False)

    for _ in range(NBUF):
        wait_write()


@jax.jit
def _embedding_lookup(idx3, table):
    b_rows = idx3.shape[0] * idx3.shape[2]
    mesh = plsc.VectorSubcoreMesh(core_axis_name="c", subcore_axis_name="s")
    out = pl.kernel(
        _gather_kernel,
        out_type=jax.ShapeDtypeStruct((T, b_rows, D), jnp.float32),
        mesh=mesh,
        scratch_types=[
            pltpu.VMEM((T, BBLK), jnp.int32),
            pltpu.VMEM((NBUF, BBLK, D), jnp.float32),
            pltpu.SemaphoreType.DMA,
            pltpu.SemaphoreType.DMA,
        ],
        compiler_params=pltpu.CompilerParams(
            use_tc_tiling_on_sc=True, needs_layout_passes=True),
    )(idx3, table)
    return jnp.transpose(out, (1, 0, 2))


def kernel(batch_original_ids, embedding_weight):
    b, t = batch_original_ids.shape
    idx3 = batch_original_ids.T.reshape(t, NW, b // NW).transpose(1, 0, 2)
    return _embedding_lookup(idx3, embedding_weight)


# speedup vs baseline: 2.6463x; 1.0016x over previous
import jax
import jax.numpy as jnp
from jax import lax
from jax.experimental import pallas as pl
from jax.experimental.pallas import tpu as pltpu, tpu_sc as plsc

NC = 2
NS = 16
NW = NC * NS
D = 128
T = 20
BBLK = 128
NBUF = 6


def _gather_kernel(idx_hbm, table_hbm, out_hbm, idx_v, bufs, gsem, wsem):
    wid = lax.axis_index("s") * NC + lax.axis_index("c")
    base = wid * BBLK

    pltpu.sync_copy(idx_hbm.at[wid], idx_v)

    def wait_gather():
        pltpu.make_async_copy(
            table_hbm.at[pl.ds(0, BBLK)], bufs.at[0], gsem).wait()

    def wait_write():
        pltpu.make_async_copy(
            bufs.at[0], out_hbm.at[0, pl.ds(base, BBLK)], wsem).wait()

    for t in range(NBUF - 1):
        pltpu.async_copy(table_hbm.at[idx_v.at[t]], bufs.at[t], gsem)

    def body(t, _):
        nt = t + NBUF - 1

        @pl.when(jnp.logical_and(nt < T, t >= 1))
        def _():
            wait_write()

        @pl.when(nt < T)
        def _():
            pltpu.async_copy(
                table_hbm.at[idx_v.at[nt]], bufs.at[lax.rem(nt, NBUF)], gsem)

        wait_gather()
        pltpu.async_copy(
            bufs.at[lax.rem(t, NBUF)], out_hbm.at[t, pl.ds(base, BBLK)], wsem)
        return ()

    lax.fori_loop(0, T, body, (), unroll=False)

    for _ in range(NBUF):
        wait_write()


@jax.jit
def _embedding_lookup(idx3, table):
    b_rows = idx3.shape[0] * idx3.shape[2]
    mesh = plsc.VectorSubcoreMesh(core_axis_name="c", subcore_axis_name="s")
    out = pl.kernel(
        _gather_kernel,
        out_type=jax.ShapeDtypeStruct((T, b_rows, D), jnp.float32),
        mesh=mesh,
        scratch_types=[
            pltpu.VMEM((T, BBLK), jnp.int32),
            pltpu.VMEM((NBUF, BBLK, D), jnp.float32),
            pltpu.SemaphoreType.DMA,
            pltpu.SemaphoreType.DMA,
        ],
        compiler_params=pltpu.CompilerParams(
            use_tc_tiling_on_sc=True, needs_layout_passes=True,
            skip_device_barrier=True),
    )(idx3, table)
    return jnp.transpose(out, (1, 0, 2))


def kernel(batch_original_ids, embedding_weight):
    b, t = batch_original_ids.shape
    idx3 = batch_original_ids.T.reshape(t, NW, b // NW).transpose(1, 0, 2)
    return _embedding_lookup(idx3, embedding_weight)
